# baseline (speedup 1.0000x reference)
import functools

import jax
import jax.numpy as jnp
from jax import lax
from jax.experimental import pallas as pl
from jax.experimental.pallas import tpu as pltpu
from jax.experimental.pallas import tpu_sc as plsc

NUM_BINS = 256
MIN_VAL = 0.0
MAX_VAL = 1.0

_L = 16
_COLS = 128
_NW = 24


def _quantize_body(x_hbm, bins_hbm, out_hbm, x_v, bins_v, out_v, sem_b, sem_x):
    nc = 2
    wid = lax.axis_index("s") * nc + lax.axis_index("c")
    rows, cols = x_v.shape

    @pl.when(wid < _NW)
    def _():
        r0 = wid * rows
        cp_b = pltpu.async_copy(bins_hbm, bins_v, sem_b)
        cp_x = pltpu.async_copy(x_hbm.at[pl.ds(r0, rows)], x_v, sem_x)
        cp_b.wait()
        cp_x.wait()

        scale = float(NUM_BINS - 1) / (MAX_VAL - MIN_VAL)
        cpr = cols // _L

        @plsc.parallel_loop(0, rows, unroll=2)
        def step(r):
            for k in range(cpr):
                v = x_v[r, pl.ds(k * _L, _L)]
                vn = jnp.minimum(jnp.maximum((v - MIN_VAL) * scale, 0.0), float(NUM_BINS - 1))
                idx = (vn + 0.5).astype(jnp.int32)
                out_v[r, pl.ds(k * _L, _L)] = plsc.load_gather(bins_v, [idx])

        pltpu.sync_copy(out_v, out_hbm.at[pl.ds(r0, rows)])


def kernel(x, bin_values):
    n = x.size
    rows = n // _COLS
    rows_per_w = rows // _NW
    assert rows_per_w * _NW == rows and rows_per_w % 8 == 0

    mesh = plsc.VectorSubcoreMesh(core_axis_name="c", subcore_axis_name="s")
    run = pl.kernel(
        _quantize_body,
        mesh=mesh,
        out_type=jax.ShapeDtypeStruct((rows, _COLS), jnp.float32),
        scratch_types=[
            pltpu.VMEM((rows_per_w, _COLS), jnp.float32),
            pltpu.VMEM((NUM_BINS,), jnp.float32),
            pltpu.VMEM((rows_per_w, _COLS), jnp.float32),
            pltpu.SemaphoreType.DMA,
            pltpu.SemaphoreType.DMA,
        ],
        compiler_params=pltpu.CompilerParams(
            needs_layout_passes=False,
            disable_bounds_checks=True,
            skip_device_barrier=True,
        ),
    )
    out = run(x.reshape(rows, _COLS), bin_values)
    return out.reshape(x.shape)


# speedup vs baseline: 1.0911x; 1.0911x over previous
import functools

import jax
import jax.numpy as jnp
from jax import lax
from jax.experimental import pallas as pl
from jax.experimental.pallas import tpu as pltpu
from jax.experimental.pallas import tpu_sc as plsc

NUM_BINS = 256
MIN_VAL = 0.0
MAX_VAL = 1.0

_L = 16
_NW = 32


def _quantize_body(x_hbm, bins_hbm, out_hbm, x_v, bins_v, out_v, sem_b, sem_x):
    nc = 2
    wid = lax.axis_index("s") * nc + lax.axis_index("c")
    rows, cols = x_v.shape
    batch, brows, _ = x_hbm.shape
    wpb = brows // rows
    b = wid // wpb
    r0 = (wid % wpb) * rows

    cp_b = pltpu.async_copy(bins_hbm, bins_v, sem_b)
    cp_x = pltpu.async_copy(x_hbm.at[b, pl.ds(r0, rows)], x_v, sem_x)
    cp_b.wait()
    cp_x.wait()

    scale = float(NUM_BINS - 1) / (MAX_VAL - MIN_VAL)
    cpr = cols // _L

    @plsc.parallel_loop(0, rows, unroll=2)
    def step(r):
        for k in range(cpr):
            v = x_v[r, pl.ds(k * _L, _L)]
            vn = jnp.minimum(jnp.maximum((v - MIN_VAL) * scale, 0.0), float(NUM_BINS - 1))
            idx = (vn + 0.5).astype(jnp.int32)
            out_v[r, pl.ds(k * _L, _L)] = plsc.load_gather(bins_v, [idx])

    pltpu.sync_copy(out_v, out_hbm.at[b, pl.ds(r0, rows)])


def kernel(x, bin_values):
    batch, brows, cols = x.shape
    assert _NW % batch == 0
    wpb = _NW // batch
    rows_per_w = brows // wpb
    assert rows_per_w * wpb == brows and rows_per_w % 8 == 0 and cols % _L == 0

    mesh = plsc.VectorSubcoreMesh(core_axis_name="c", subcore_axis_name="s")
    run = pl.kernel(
        _quantize_body,
        mesh=mesh,
        out_type=jax.ShapeDtypeStruct((batch, brows, cols), jnp.float32),
        scratch_types=[
            pltpu.VMEM((rows_per_w, cols), jnp.float32),
            pltpu.VMEM((NUM_BINS,), jnp.float32),
            pltpu.VMEM((rows_per_w, cols), jnp.float32),
            pltpu.SemaphoreType.DMA,
            pltpu.SemaphoreType.DMA,
        ],
        compiler_params=pltpu.CompilerParams(
            needs_layout_passes=False,
            disable_bounds_checks=True,
            skip_device_barrier=True,
        ),
    )
    return run(x, bin_values)
